# baseline (speedup 1.0000x reference)
import jax
import jax.numpy as jnp
from jax import lax
from jax.experimental import pallas as pl
from jax.experimental.pallas import tpu as pltpu
from jax.experimental.pallas import tpu_sc as plsc

N_NODES = 10000
N_EDGES = 160000
VOCAB = 1000
D_IN = 1024
D_H = 256
HALF = 128
NC = 2
NS = 16
LANES = 16
KE = 112
EPT = N_EDGES // NS
EPT_P = 10080
KCH = EPT_P // KE
KN = 80
NBLK = N_NODES // KN
BPT = (NBLK + NS - 1) // NS
ACC_ROWS = 10008
DEG_PAD = NS * 640

f32 = jnp.float32
i32 = jnp.int32


def _proj1_body(emb_ref, ws_ref, wn_ref, ps0_ref, ps1_ref, pn0_ref, pn1_ref):
    e = emb_ref[...]
    ps = jnp.dot(e, ws_ref[...], preferred_element_type=f32)
    pn = jnp.dot(e, wn_ref[...], preferred_element_type=f32)
    ps0_ref[...] = ps[:, :HALF]
    ps1_ref[...] = ps[:, HALF:]
    pn0_ref[...] = pn[:, :HALF]
    pn1_ref[...] = pn[:, HALF:]


def _project_l1(emb, W_self1, W_neigh1):
    out = jax.ShapeDtypeStruct((VOCAB, HALF), f32)
    return pl.pallas_call(_proj1_body, out_shape=(out,) * 4)(
        emb, W_self1, W_neigh1)


def _proj2_body(h1a_ref, h1b_ref, ws_ref, wn_ref, s0_ref, s1_ref, g0_ref, g1_ref):
    a = h1a_ref[...]
    b = h1b_ref[...]
    ws = ws_ref[...]
    wn = wn_ref[...]
    s = (jnp.dot(a, ws[:HALF, :], preferred_element_type=f32)
         + jnp.dot(b, ws[HALF:, :], preferred_element_type=f32))
    g = (jnp.dot(a, wn[:HALF, :], preferred_element_type=f32)
         + jnp.dot(b, wn[HALF:, :], preferred_element_type=f32))
    s0_ref[...] = s[:, :HALF]
    s1_ref[...] = s[:, HALF:]
    g0_ref[...] = g[:, :HALF]
    g1_ref[...] = g[:, HALF:]


def _project_l2(h1a, h1b, W_self2, W_neigh2):
    R = 1000
    bs_in = pl.BlockSpec((R, HALF), lambda i: (i, 0))
    bs_w = pl.BlockSpec((D_H, D_H), lambda i: (0, 0))
    bs_out = pl.BlockSpec((R, HALF), lambda i: (i, 0))
    out = jax.ShapeDtypeStruct((N_NODES, HALF), f32)
    return pl.pallas_call(
        _proj2_body,
        grid=(N_NODES // R,),
        in_specs=[bs_in, bs_in, bs_w, bs_w],
        out_specs=(bs_out,) * 4,
        out_shape=(out,) * 4,
    )(h1a, h1b, W_self2, W_neigh2)


def _zero_vmem_2d(ref, nrows):
    def zrow(i, carry):
        for jj in range(HALF // LANES):
            ref[i, pl.ds(jj * LANES, LANES)] = jnp.zeros((LANES,), f32)
        return carry
    lax.fori_loop(0, nrows, zrow, None)


def _fill_vmem_1d(ref, n, value):
    def fill(i, carry):
        ref[pl.ds(i * LANES, LANES)] = jnp.full((LANES,), value, f32)
        return carry
    lax.fori_loop(0, n // LANES, fill, None)


def _agg_pipe(tab_ref, isrc, idst, rows, rows2, acc, sem_a, sem_b,
              deg=None, rdbuf=None, sem_d=None):
    def gidx(k):
        return isrc.at[pl.ds(k * KE, KE)]

    def didx(k):
        return idst.at[pl.ds(k * KE, KE)]

    def scat(buf, k):
        pltpu.sync_copy(buf, acc.at[didx(k)], add=True)
        if deg is not None:
            pltpu.async_copy(rdbuf, deg.at[didx(k)], sem_d, add=True)

    pltpu.async_copy(tab_ref.at[gidx(0)], rows, sem_a)

    def body(kk, carry):
        k0 = 2 * kk
        k1 = k0 + 1
        pltpu.make_async_copy(tab_ref.at[gidx(k0)], rows, sem_a).wait()
        pltpu.async_copy(tab_ref.at[gidx(k1)], rows2, sem_b)
        scat(rows, k0)
        pltpu.make_async_copy(tab_ref.at[gidx(k1)], rows2, sem_b).wait()

        @pl.when(k0 + 2 < KCH)
        def _():
            pltpu.async_copy(tab_ref.at[gidx(k0 + 2)], rows, sem_a)
        scat(rows2, k1)
        return carry
    lax.fori_loop(0, KCH // 2, body, None)
    if KCH % 2:
        pltpu.make_async_copy(tab_ref.at[gidx(KCH - 1)], rows, sem_a).wait()
        scat(rows, KCH - 1)
    if deg is not None:
        def drain(k, carry):
            pltpu.make_async_copy(rdbuf, deg.at[didx(0)], sem_d).wait()
            return carry
        lax.fori_loop(0, KCH, drain, None)


def _sc1_body(node_hbm, src_hbm, dst_hbm, ps0, ps1, pn0, pn1, b1_hbm,
              h1a, h1b, rdeg_out, ta, tb,
              acc, deg, isrc, idst, rows, rows2, degblk,
              b1buf, nidx, rdbuf, sem_a, sem_b, sem_d):
    c = lax.axis_index("c")
    s = lax.axis_index("s")

    _zero_vmem_2d(rows, KN)
    _fill_vmem_1d(rdbuf, KE, 0.0)

    def zblk(q, carry):
        b = s * BPT + q

        @pl.when(b < NBLK)
        def _():
            pltpu.sync_copy(rows.at[pl.ds(0, KN)], acc.at[pl.ds(b * KN, KN)])
        return carry
    lax.fori_loop(0, BPT, zblk, None)
    for q in range(8):
        pltpu.sync_copy(rdbuf.at[pl.ds(0, KN)],
                        deg.at[pl.ds(s * 640 + q * KN, KN)])
    _fill_vmem_1d(rdbuf, KE, 1.0)

    def tbuild(pn_ref, t_ref):
        def blk(q, carry):
            b = s * BPT + q

            @pl.when(b < NBLK)
            def _():
                base = b * KN
                pltpu.sync_copy(node_hbm.at[pl.ds(base, KN)], nidx)
                pltpu.async_copy(pn_ref.at[nidx], rows2.at[pl.ds(0, KN)],
                                 sem_a).wait()
                pltpu.sync_copy(rows2.at[pl.ds(0, KN)],
                                t_ref.at[pl.ds(base, KN)])
            return carry
        lax.fori_loop(0, BPT, blk, None)

    @pl.when(c == 0)
    def _():
        tbuild(pn0, ta)

    @pl.when(c == 1)
    def _():
        tbuild(pn1, tb)

    pltpu.sync_copy(src_hbm.at[pl.ds(s * EPT_P, EPT_P)], isrc)
    pltpu.sync_copy(dst_hbm.at[pl.ds(s * EPT_P, EPT_P)], idst)
    plsc.subcore_barrier()

    @pl.when(c == 0)
    def _():
        _agg_pipe(ta, isrc, idst, rows, rows2, acc, sem_a, sem_b, deg, rdbuf,
                  sem_d)

    @pl.when(c == 1)
    def _():
        _agg_pipe(tb, isrc, idst, rows, rows2, acc, sem_a, sem_b, deg, rdbuf,
                  sem_d)

    plsc.subcore_barrier()

    pltpu.sync_copy(b1_hbm.at[pl.ds(c * HALF, HALF)], b1buf)

    def combine(ps_ref, hout_ref, do_rdeg):
        def blk(j, carry):
            b = s * BPT + j

            @pl.when(b < NBLK)
            def _():
                base = b * KN
                pltpu.sync_copy(node_hbm.at[pl.ds(base, KN)], nidx)
                pltpu.async_copy(ps_ref.at[nidx], rows.at[pl.ds(0, KN)],
                                 sem_a).wait()
                pltpu.sync_copy(acc.at[pl.ds(base, KN)],
                                rows2.at[pl.ds(0, KN)])
                pltpu.sync_copy(deg.at[pl.ds(base, KN)], degblk)

                def tfn(t, carry2):
                    dvec = degblk[pl.ds(t * LANES, LANES)]
                    rdvec = 1.0 / jnp.maximum(dvec, 1.0)
                    for l in range(LANES):
                        i = t * LANES + l
                        rd = lax.broadcast_in_dim(
                            lax.slice(rdvec, (l,), (l + 1,)), (LANES,), (0,))
                        for jj in range(HALF // LANES):
                            sl = pl.ds(jj * LANES, LANES)
                            v = rows[i, sl] + rows2[i, sl] * rd + b1buf[sl]
                            rows[i, sl] = jnp.maximum(v, 0.0)
                    return carry2
                lax.fori_loop(0, KN // LANES, tfn, None)
                pltpu.sync_copy(rows.at[pl.ds(0, KN)],
                                hout_ref.at[pl.ds(base, KN)])
                if do_rdeg:
                    def rv(t, carry2):
                        sl = pl.ds(t * LANES, LANES)
                        rdbuf[sl] = 1.0 / jnp.maximum(degblk[sl], 1.0)
                        return carry2
                    lax.fori_loop(0, KN // LANES, rv, None)
                    pltpu.sync_copy(rdbuf.at[pl.ds(0, KN)],
                                    rdeg_out.at[pl.ds(base, KN)])
            return carry
        lax.fori_loop(0, BPT, blk, None)

    @pl.when(c == 0)
    def _():
        combine(ps0, h1a, True)

    @pl.when(c == 1)
    def _():
        combine(ps1, h1b, False)


def _sc_layer1(node, srcp, dstp, ps0, ps1, pn0, pn1, b1):
    mesh = plsc.VectorSubcoreMesh(core_axis_name="c", subcore_axis_name="s",
                                  num_cores=NC, num_subcores=NS)
    kern = pl.kernel(
        _sc1_body,
        out_type=[jax.ShapeDtypeStruct((N_NODES, HALF), f32),
                  jax.ShapeDtypeStruct((N_NODES, HALF), f32),
                  jax.ShapeDtypeStruct((N_NODES,), f32),
                  jax.ShapeDtypeStruct((N_NODES, HALF), f32),
                  jax.ShapeDtypeStruct((N_NODES, HALF), f32)],
        mesh=mesh,
        scratch_types=[
            pltpu.VMEM_SHARED((ACC_ROWS, HALF), f32),
            pltpu.VMEM_SHARED((DEG_PAD,), f32),
            pltpu.VMEM((EPT_P,), i32),
            pltpu.VMEM((EPT_P,), i32),
            pltpu.VMEM((KE, HALF), f32),
            pltpu.VMEM((KE, HALF), f32),
            pltpu.VMEM((KN,), f32),
            pltpu.VMEM((HALF,), f32),
            pltpu.VMEM((KN,), i32),
            pltpu.VMEM((KE,), f32),
            pltpu.SemaphoreType.DMA,
            pltpu.SemaphoreType.DMA,
            pltpu.SemaphoreType.DMA,
        ],
    )
    h1a, h1b, rdeg, _ta, _tb = kern(node, srcp, dstp, ps0, ps1, pn0, pn1, b1)
    return h1a, h1b, rdeg


def _sc2_body(src_hbm, dst_hbm, s2a, s2b, g2a, g2b, rdeg_hbm, b2_hbm,
              h2_out,
              acc, isrc, idst, rows, rows2, degblk, b2buf, sem_a, sem_b):
    c = lax.axis_index("c")
    s = lax.axis_index("s")

    _zero_vmem_2d(rows, KN)

    def zblk(q, carry):
        b = s * BPT + q

        @pl.when(b < NBLK)
        def _():
            pltpu.sync_copy(rows.at[pl.ds(0, KN)], acc.at[pl.ds(b * KN, KN)])
        return carry
    lax.fori_loop(0, BPT, zblk, None)

    pltpu.sync_copy(src_hbm.at[pl.ds(s * EPT_P, EPT_P)], isrc)
    pltpu.sync_copy(dst_hbm.at[pl.ds(s * EPT_P, EPT_P)], idst)
    plsc.subcore_barrier()

    @pl.when(c == 0)
    def _():
        _agg_pipe(g2a, isrc, idst, rows, rows2, acc, sem_a, sem_b)

    @pl.when(c == 1)
    def _():
        _agg_pipe(g2b, isrc, idst, rows, rows2, acc, sem_a, sem_b)

    plsc.subcore_barrier()

    pltpu.sync_copy(b2_hbm.at[pl.ds(c * HALF, HALF)], b2buf)

    def combine(s_ref):
        def blk(j, carry):
            b = s * BPT + j

            @pl.when(b < NBLK)
            def _():
                base = b * KN
                pltpu.sync_copy(s_ref.at[pl.ds(base, KN)],
                                rows.at[pl.ds(0, KN)])
                pltpu.sync_copy(acc.at[pl.ds(base, KN)],
                                rows2.at[pl.ds(0, KN)])
                pltpu.sync_copy(rdeg_hbm.at[pl.ds(base, KN)], degblk)

                def tfn(t, carry2):
                    rdvec = degblk[pl.ds(t * LANES, LANES)]
                    for l in range(LANES):
                        i = t * LANES + l
                        rd = lax.broadcast_in_dim(
                            lax.slice(rdvec, (l,), (l + 1,)), (LANES,), (0,))
                        for jj in range(HALF // LANES):
                            sl = pl.ds(jj * LANES, LANES)
                            rows[i, sl] = (rows[i, sl] + rows2[i, sl] * rd
                                           + b2buf[sl])
                    return carry2
                lax.fori_loop(0, KN // LANES, tfn, None)
                pltpu.sync_copy(
                    rows.at[pl.ds(0, KN)],
                    h2_out.at[pl.ds(base, KN), pl.ds(c * HALF, HALF)])
            return carry
        lax.fori_loop(0, BPT, blk, None)

    @pl.when(c == 0)
    def _():
        combine(s2a)

    @pl.when(c == 1)
    def _():
        combine(s2b)


def _sc_layer2(srcp, dstp, s2a, s2b, g2a, g2b, rdeg, b2):
    mesh = plsc.VectorSubcoreMesh(core_axis_name="c", subcore_axis_name="s",
                                  num_cores=NC, num_subcores=NS)
    kern = pl.kernel(
        _sc2_body,
        out_type=jax.ShapeDtypeStruct((N_NODES, D_H), f32),
        mesh=mesh,
        scratch_types=[
            pltpu.VMEM_SHARED((ACC_ROWS, HALF), f32),
            pltpu.VMEM((EPT_P,), i32),
            pltpu.VMEM((EPT_P,), i32),
            pltpu.VMEM((KE, HALF), f32),
            pltpu.VMEM((KE, HALF), f32),
            pltpu.VMEM((KN,), f32),
            pltpu.VMEM((HALF,), f32),
            pltpu.SemaphoreType.DMA,
            pltpu.SemaphoreType.DMA,
        ],
    )
    return kern(srcp, dstp, s2a, s2b, g2a, g2b, rdeg, b2)


def kernel(node, edge_index, emb, W_self1, W_neigh1, b1, W_self2, W_neigh2, b2):
    pad = EPT_P - EPT
    src2 = edge_index[0].reshape(NS, EPT)
    dst2 = edge_index[1].reshape(NS, EPT)
    srcp = jnp.pad(src2, ((0, 0), (0, pad))).reshape(NS * EPT_P)
    dstp = jnp.pad(dst2, ((0, 0), (0, pad)),
                   constant_values=N_NODES).reshape(NS * EPT_P)
    ps0, ps1, pn0, pn1 = _project_l1(emb, W_self1, W_neigh1)
    h1a, h1b, rdeg = _sc_layer1(node, srcp, dstp, ps0, ps1, pn0, pn1, b1)
    s2a, s2b, g2a, g2b = _project_l2(h1a, h1b, W_self2, W_neigh2)
    return _sc_layer2(srcp, dstp, s2a, s2b, g2a, g2b, rdeg, b2)


# speedup vs baseline: 1.1867x; 1.1867x over previous
import jax
import jax.numpy as jnp
from jax import lax
from jax.experimental import pallas as pl
from jax.experimental.pallas import tpu as pltpu
from jax.experimental.pallas import tpu_sc as plsc

N_NODES = 10000
N_EDGES = 160000
VOCAB = 1000
D_IN = 1024
D_H = 256
HALF = 128
NC = 2
NS = 16
LANES = 16
K = 80
EPT = N_EDGES // NS
KCH = EPT // K
NBLK = N_NODES // K
BPT = (NBLK + NS - 1) // NS
ZROWS = 125
DEG_PAD = NS * 640

f32 = jnp.float32
i32 = jnp.int32


def _proj1_body(emb_ref, ws_ref, wn_ref, ps0_ref, ps1_ref, pn0_ref, pn1_ref):
    e = emb_ref[...]
    ps = jnp.dot(e, ws_ref[...], preferred_element_type=f32)
    pn = jnp.dot(e, wn_ref[...], preferred_element_type=f32)
    ps0_ref[...] = ps[:, :HALF]
    ps1_ref[...] = ps[:, HALF:]
    pn0_ref[...] = pn[:, :HALF]
    pn1_ref[...] = pn[:, HALF:]


def _project_l1(emb, W_self1, W_neigh1):
    out = jax.ShapeDtypeStruct((VOCAB, HALF), f32)
    return pl.pallas_call(_proj1_body, out_shape=(out,) * 4)(
        emb, W_self1, W_neigh1)


def _proj2_body(h1a_ref, h1b_ref, ws_ref, wn_ref, s0_ref, s1_ref, g0_ref, g1_ref):
    a = h1a_ref[...]
    b = h1b_ref[...]
    ws = ws_ref[...]
    wn = wn_ref[...]
    s = (jnp.dot(a, ws[:HALF, :], preferred_element_type=f32)
         + jnp.dot(b, ws[HALF:, :], preferred_element_type=f32))
    g = (jnp.dot(a, wn[:HALF, :], preferred_element_type=f32)
         + jnp.dot(b, wn[HALF:, :], preferred_element_type=f32))
    s0_ref[...] = s[:, :HALF]
    s1_ref[...] = s[:, HALF:]
    g0_ref[...] = g[:, :HALF]
    g1_ref[...] = g[:, HALF:]


def _project_l2(h1a, h1b, W_self2, W_neigh2):
    R = 1000
    bs_in = pl.BlockSpec((R, HALF), lambda i: (i, 0))
    bs_w = pl.BlockSpec((D_H, D_H), lambda i: (0, 0))
    bs_out = pl.BlockSpec((R, HALF), lambda i: (i, 0))
    out = jax.ShapeDtypeStruct((N_NODES, HALF), f32)
    return pl.pallas_call(
        _proj2_body,
        grid=(N_NODES // R,),
        in_specs=[bs_in, bs_in, bs_w, bs_w],
        out_specs=(bs_out,) * 4,
        out_shape=(out,) * 4,
    )(h1a, h1b, W_self2, W_neigh2)


def _zero_vmem_2d(ref, nrows):
    def zrow(i, carry):
        for jj in range(HALF // LANES):
            ref[i, pl.ds(jj * LANES, LANES)] = jnp.zeros((LANES,), f32)
        return carry
    lax.fori_loop(0, nrows, zrow, None)


def _fill_vmem_1d(ref, n, value):
    def fill(i, carry):
        ref[pl.ds(i * LANES, LANES)] = jnp.full((LANES,), value, f32)
        return carry
    lax.fori_loop(0, n // LANES, fill, None)


def _agg_pipe(tab_ref, isrc, idst, rows, rows2, acc, sem_a, sem_b,
              deg=None, rdbuf=None, sem_d=None,
              node_sp=None, na=None, nb=None):
    def gidx(k):
        return isrc.at[pl.ds(k * K, K)]

    def didx(k):
        return idst.at[pl.ds(k * K, K)]

    def scat(buf, k):
        pltpu.sync_copy(buf, acc.at[didx(k)], add=True)
        if deg is not None:
            pltpu.async_copy(rdbuf, deg.at[didx(k)], sem_d, add=True)

    def ng(k, nbuf):
        pltpu.sync_copy(node_sp.at[gidx(k)], nbuf)

    def rg(k, nbuf, buf, sem):
        if node_sp is None:
            pltpu.async_copy(tab_ref.at[gidx(k)], buf, sem)
        else:
            pltpu.async_copy(tab_ref.at[nbuf], buf, sem)

    def rwait(buf, sem):
        pltpu.make_async_copy(tab_ref.at[gidx(0)], buf, sem).wait()

    if node_sp is not None:
        ng(0, na)
    rg(0, na, rows, sem_a)

    def body(kk, carry):
        k0 = 2 * kk
        k1 = k0 + 1
        if node_sp is not None:
            ng(k1, nb)
        rwait(rows, sem_a)
        rg(k1, nb, rows2, sem_b)
        scat(rows, k0)
        if node_sp is not None:
            ng(k0 + 2, na)
        rwait(rows2, sem_b)
        rg(k0 + 2, na, rows, sem_a)
        scat(rows2, k1)
        return carry
    lax.fori_loop(0, (KCH - 1) // 2, body, None)
    rwait(rows, sem_a)
    scat(rows, KCH - 1)
    if deg is not None:
        def drain(k, carry):
            pltpu.make_async_copy(rdbuf, deg.at[didx(0)], sem_d).wait()
            return carry
        lax.fori_loop(0, KCH, drain, None)


def _sc1_body(node_hbm, src_hbm, dst_hbm, ps0, ps1, pn0, pn1, b1_hbm,
              h1a, h1b, rdeg_out,
              acc, deg, pn_sp, node_sp, isrc, idst, na, nb,
              rows, rows2, degblk, b1buf, nidx, rdbuf, sem_a, sem_b, sem_d):
    c = lax.axis_index("c")
    s = lax.axis_index("s")

    _zero_vmem_2d(rows, K)
    _fill_vmem_1d(rdbuf, K, 0.0)
    def zblk(q, carry):
        b = s * BPT + q

        @pl.when(b < NBLK)
        def _():
            pltpu.sync_copy(rows, acc.at[pl.ds(b * K, K)])
        return carry
    lax.fori_loop(0, BPT, zblk, None)
    for q in range(8):
        pltpu.sync_copy(rdbuf, deg.at[pl.ds(s * 640 + q * K, K)])
    _fill_vmem_1d(rdbuf, K, 1.0)

    @pl.when(s == 0)
    def _():
        @pl.when(c == 0)
        def _():
            pltpu.sync_copy(pn0, pn_sp)

        @pl.when(c == 1)
        def _():
            pltpu.sync_copy(pn1, pn_sp)
        pltpu.sync_copy(node_hbm, node_sp)

    pltpu.sync_copy(src_hbm.at[pl.ds(s * EPT, EPT)], isrc)
    pltpu.sync_copy(dst_hbm.at[pl.ds(s * EPT, EPT)], idst)
    plsc.subcore_barrier()

    _agg_pipe(pn_sp, isrc, idst, rows, rows2, acc, sem_a, sem_b, deg, rdbuf,
              sem_d, node_sp, na, nb)

    plsc.subcore_barrier()

    pltpu.sync_copy(b1_hbm.at[pl.ds(c * HALF, HALF)], b1buf)

    def combine(ps_ref, hout_ref, do_rdeg):
        def blk(j, carry):
            b = s * BPT + j

            @pl.when(b < NBLK)
            def _():
                base = b * K
                pltpu.sync_copy(node_hbm.at[pl.ds(base, K)], nidx)
                pltpu.async_copy(ps_ref.at[nidx], rows, sem_a).wait()
                pltpu.sync_copy(acc.at[pl.ds(base, K)], rows2)
                pltpu.sync_copy(deg.at[pl.ds(base, K)], degblk)

                def tfn(t, carry2):
                    dvec = degblk[pl.ds(t * LANES, LANES)]
                    rdvec = 1.0 / jnp.maximum(dvec, 1.0)
                    for l in range(LANES):
                        i = t * LANES + l
                        rd = lax.broadcast_in_dim(
                            lax.slice(rdvec, (l,), (l + 1,)), (LANES,), (0,))
                        for jj in range(HALF // LANES):
                            sl = pl.ds(jj * LANES, LANES)
                            v = rows[i, sl] + rows2[i, sl] * rd + b1buf[sl]
                            rows[i, sl] = jnp.maximum(v, 0.0)
                    return carry2
                lax.fori_loop(0, K // LANES, tfn, None)
                pltpu.sync_copy(rows, hout_ref.at[pl.ds(base, K)])
                if do_rdeg:
                    def rv(i, carry2):
                        sl = pl.ds(i * LANES, LANES)
                        rdbuf[sl] = 1.0 / jnp.maximum(degblk[sl], 1.0)
                        return carry2
                    lax.fori_loop(0, K // LANES, rv, None)
                    pltpu.sync_copy(rdbuf, rdeg_out.at[pl.ds(base, K)])
            return carry
        lax.fori_loop(0, BPT, blk, None)

    @pl.when(c == 0)
    def _():
        combine(ps0, h1a, True)

    @pl.when(c == 1)
    def _():
        combine(ps1, h1b, False)


def _sc_layer1(node, srcf, dstf, ps0, ps1, pn0, pn1, b1):
    mesh = plsc.VectorSubcoreMesh(core_axis_name="c", subcore_axis_name="s",
                                  num_cores=NC, num_subcores=NS)
    kern = pl.kernel(
        _sc1_body,
        out_type=[jax.ShapeDtypeStruct((N_NODES, HALF), f32),
                  jax.ShapeDtypeStruct((N_NODES, HALF), f32),
                  jax.ShapeDtypeStruct((N_NODES,), f32)],
        mesh=mesh,
        scratch_types=[
            pltpu.VMEM_SHARED((N_NODES, HALF), f32),
            pltpu.VMEM_SHARED((DEG_PAD,), f32),
            pltpu.VMEM_SHARED((VOCAB, HALF), f32),
            pltpu.VMEM_SHARED((N_NODES,), i32),
            pltpu.VMEM((EPT,), i32),
            pltpu.VMEM((EPT,), i32),
            pltpu.VMEM((K,), i32),
            pltpu.VMEM((K,), i32),
            pltpu.VMEM((K, HALF), f32),
            pltpu.VMEM((K, HALF), f32),
            pltpu.VMEM((K,), f32),
            pltpu.VMEM((HALF,), f32),
            pltpu.VMEM((K,), i32),
            pltpu.VMEM((K,), f32),
            pltpu.SemaphoreType.DMA,
            pltpu.SemaphoreType.DMA,
            pltpu.SemaphoreType.DMA,
        ],
    )
    return kern(node, srcf, dstf, ps0, ps1, pn0, pn1, b1)


def _sc2_body(src_hbm, dst_hbm, s2a, s2b, g2a, g2b, rdeg_hbm, b2_hbm,
              h2_out,
              acc, isrc, idst, rows, rows2, degblk, b2buf, sem_a, sem_b):
    c = lax.axis_index("c")
    s = lax.axis_index("s")

    _zero_vmem_2d(rows, K)
    def zblk(q, carry):
        b = s * BPT + q

        @pl.when(b < NBLK)
        def _():
            pltpu.sync_copy(rows, acc.at[pl.ds(b * K, K)])
        return carry
    lax.fori_loop(0, BPT, zblk, None)
    plsc.subcore_barrier()

    pltpu.sync_copy(src_hbm.at[pl.ds(s * EPT, EPT)], isrc)
    pltpu.sync_copy(dst_hbm.at[pl.ds(s * EPT, EPT)], idst)

    @pl.when(c == 0)
    def _():
        _agg_pipe(g2a, isrc, idst, rows, rows2, acc, sem_a, sem_b)

    @pl.when(c == 1)
    def _():
        _agg_pipe(g2b, isrc, idst, rows, rows2, acc, sem_a, sem_b)

    plsc.subcore_barrier()

    pltpu.sync_copy(b2_hbm.at[pl.ds(c * HALF, HALF)], b2buf)

    def combine(s_ref):
        def blk(j, carry):
            b = s * BPT + j

            @pl.when(b < NBLK)
            def _():
                base = b * K
                pltpu.sync_copy(s_ref.at[pl.ds(base, K)], rows)
                pltpu.sync_copy(acc.at[pl.ds(base, K)], rows2)
                pltpu.sync_copy(rdeg_hbm.at[pl.ds(base, K)], degblk)

                def tfn(t, carry2):
                    rdvec = degblk[pl.ds(t * LANES, LANES)]
                    for l in range(LANES):
                        i = t * LANES + l
                        rd = lax.broadcast_in_dim(
                            lax.slice(rdvec, (l,), (l + 1,)), (LANES,), (0,))
                        for jj in range(HALF // LANES):
                            sl = pl.ds(jj * LANES, LANES)
                            rows[i, sl] = (rows[i, sl] + rows2[i, sl] * rd
                                           + b2buf[sl])
                    return carry2
                lax.fori_loop(0, K // LANES, tfn, None)
                pltpu.sync_copy(
                    rows, h2_out.at[pl.ds(base, K), pl.ds(c * HALF, HALF)])
            return carry
        lax.fori_loop(0, BPT, blk, None)

    @pl.when(c == 0)
    def _():
        combine(s2a)

    @pl.when(c == 1)
    def _():
        combine(s2b)


def _sc_layer2(srcf, dstf, s2a, s2b, g2a, g2b, rdeg, b2):
    mesh = plsc.VectorSubcoreMesh(core_axis_name="c", subcore_axis_name="s",
                                  num_cores=NC, num_subcores=NS)
    kern = pl.kernel(
        _sc2_body,
        out_type=jax.ShapeDtypeStruct((N_NODES, D_H), f32),
        mesh=mesh,
        scratch_types=[
            pltpu.VMEM_SHARED((N_NODES, HALF), f32),
            pltpu.VMEM((EPT,), i32),
            pltpu.VMEM((EPT,), i32),
            pltpu.VMEM((K, HALF), f32),
            pltpu.VMEM((K, HALF), f32),
            pltpu.VMEM((K,), f32),
            pltpu.VMEM((HALF,), f32),
            pltpu.SemaphoreType.DMA,
            pltpu.SemaphoreType.DMA,
        ],
    )
    return kern(srcf, dstf, s2a, s2b, g2a, g2b, rdeg, b2)


def kernel(node, edge_index, emb, W_self1, W_neigh1, b1, W_self2, W_neigh2, b2):
    srcf = edge_index[0]
    dstf = edge_index[1]
    ps0, ps1, pn0, pn1 = _project_l1(emb, W_self1, W_neigh1)
    h1a, h1b, rdeg = _sc_layer1(node, srcf, dstf, ps0, ps1, pn0, pn1, b1)
    s2a, s2b, g2a, g2b = _project_l2(h1a, h1b, W_self2, W_neigh2)
    return _sc_layer2(srcf, dstf, s2a, s2b, g2a, g2b, rdeg, b2)


# speedup vs baseline: 1.2251x; 1.0324x over previous
import jax
import jax.numpy as jnp
from jax import lax
from jax.experimental import pallas as pl
from jax.experimental.pallas import tpu as pltpu
from jax.experimental.pallas import tpu_sc as plsc

N_NODES = 10000
N_EDGES = 160000
VOCAB = 1000
D_IN = 1024
D_H = 256
HALF = 128
NC = 2
NS = 16
LANES = 16
K = 80
EPT = N_EDGES // NS
KCH = EPT // K
NBLK = N_NODES // K
BPT = (NBLK + NS - 1) // NS
ZROWS = 125
DEG_PAD = NS * 640

f32 = jnp.float32
i32 = jnp.int32


def _proj1_body(emb_ref, ws_ref, wn_ref, ps0_ref, ps1_ref, pn0_ref, pn1_ref):
    e = emb_ref[...]
    ps = jnp.dot(e, ws_ref[...], preferred_element_type=f32)
    pn = jnp.dot(e, wn_ref[...], preferred_element_type=f32)
    ps0_ref[...] = ps[:, :HALF]
    ps1_ref[...] = ps[:, HALF:]
    pn0_ref[...] = pn[:, :HALF]
    pn1_ref[...] = pn[:, HALF:]


def _project_l1(emb, W_self1, W_neigh1):
    out = jax.ShapeDtypeStruct((VOCAB, HALF), f32)
    return pl.pallas_call(_proj1_body, out_shape=(out,) * 4)(
        emb, W_self1, W_neigh1)


def _proj2_body(h1a_ref, h1b_ref, ws_ref, wn_ref, s0_ref, s1_ref, g0_ref, g1_ref):
    a = h1a_ref[...]
    b = h1b_ref[...]
    ws = ws_ref[...]
    wn = wn_ref[...]
    s = (jnp.dot(a, ws[:HALF, :], preferred_element_type=f32)
         + jnp.dot(b, ws[HALF:, :], preferred_element_type=f32))
    g = (jnp.dot(a, wn[:HALF, :], preferred_element_type=f32)
         + jnp.dot(b, wn[HALF:, :], preferred_element_type=f32))
    s0_ref[...] = s[:, :HALF]
    s1_ref[...] = s[:, HALF:]
    g0_ref[...] = g[:, :HALF]
    g1_ref[...] = g[:, HALF:]


def _project_l2(h1a, h1b, W_self2, W_neigh2):
    R = 1000
    bs_in = pl.BlockSpec((R, HALF), lambda i: (i, 0))
    bs_w = pl.BlockSpec((D_H, D_H), lambda i: (0, 0))
    bs_out = pl.BlockSpec((R, HALF), lambda i: (i, 0))
    out = jax.ShapeDtypeStruct((N_NODES, HALF), f32)
    return pl.pallas_call(
        _proj2_body,
        grid=(N_NODES // R,),
        in_specs=[bs_in, bs_in, bs_w, bs_w],
        out_specs=(bs_out,) * 4,
        out_shape=(out,) * 4,
    )(h1a, h1b, W_self2, W_neigh2)


def _zero_vmem_2d(ref, nrows):
    def zrow(i, carry):
        for jj in range(HALF // LANES):
            ref[i, pl.ds(jj * LANES, LANES)] = jnp.zeros((LANES,), f32)
        return carry
    lax.fori_loop(0, nrows, zrow, None)


def _fill_vmem_1d(ref, n, value):
    def fill(i, carry):
        ref[pl.ds(i * LANES, LANES)] = jnp.full((LANES,), value, f32)
        return carry
    lax.fori_loop(0, n // LANES, fill, None)


def _agg_pipe(tab_ref, isrc, idst, rows, rows2, acc, sem_a, sem_b,
              deg=None, rdbuf=None, sem_d=None,
              node_sp=None, na=None, nb=None):
    def gidx(k):
        return isrc.at[pl.ds(k * K, K)]

    def didx(k):
        return idst.at[pl.ds(k * K, K)]

    def scat(buf, k):
        pltpu.sync_copy(buf, acc.at[didx(k)], add=True)
        if deg is not None:
            pltpu.async_copy(rdbuf, deg.at[didx(k)], sem_d, add=True)

    def ng(k, nbuf):
        pltpu.sync_copy(node_sp.at[gidx(k)], nbuf)

    def rg(k, nbuf, buf, sem):
        if node_sp is None:
            pltpu.async_copy(tab_ref.at[gidx(k)], buf, sem)
        else:
            pltpu.async_copy(tab_ref.at[nbuf], buf, sem)

    def rwait(buf, sem):
        pltpu.make_async_copy(tab_ref.at[gidx(0)], buf, sem).wait()

    if node_sp is not None:
        ng(0, na)
    rg(0, na, rows, sem_a)

    def body(kk, carry):
        k0 = 2 * kk
        k1 = k0 + 1
        if node_sp is not None:
            ng(k1, nb)
        rwait(rows, sem_a)
        rg(k1, nb, rows2, sem_b)
        scat(rows, k0)
        if node_sp is not None:
            ng(k0 + 2, na)
        rwait(rows2, sem_b)
        rg(k0 + 2, na, rows, sem_a)
        scat(rows2, k1)
        return carry
    lax.fori_loop(0, (KCH - 1) // 2, body, None)
    rwait(rows, sem_a)
    scat(rows, KCH - 1)
    if deg is not None:
        def drain(k, carry):
            pltpu.make_async_copy(rdbuf, deg.at[didx(0)], sem_d).wait()
            return carry
        lax.fori_loop(0, KCH, drain, None)


def _sc1_body(node_hbm, src_hbm, dst_hbm, ps0, ps1, pn0, pn1, b1_hbm,
              h1a, h1b, rdeg_out,
              acc, deg, pn_sp, node_sp, isrc, idst, na, nb,
              rows, rows2, degblk, b1buf, nidx, rdbuf, sem_a, sem_b, sem_c,
              sem_d):
    c = lax.axis_index("c")
    s = lax.axis_index("s")

    _zero_vmem_2d(rows, K)
    _fill_vmem_1d(rdbuf, K, 0.0)
    def zblk(q, carry):
        b = s * BPT + q

        @pl.when(b < NBLK)
        def _():
            pltpu.async_copy(rows, acc.at[pl.ds(b * K, K)], sem_a)
        return carry
    lax.fori_loop(0, BPT, zblk, None)
    for q in range(8):
        pltpu.async_copy(rdbuf, deg.at[pl.ds(s * 640 + q * K, K)], sem_b)
    def zdrain(q, carry):
        b = s * BPT + q

        @pl.when(b < NBLK)
        def _():
            pltpu.make_async_copy(rows, acc.at[pl.ds(0, K)], sem_a).wait()
        return carry
    lax.fori_loop(0, BPT, zdrain, None)
    for q in range(8):
        pltpu.make_async_copy(rdbuf, deg.at[pl.ds(0, K)], sem_b).wait()
    _fill_vmem_1d(rdbuf, K, 1.0)

    @pl.when(s == 0)
    def _():
        @pl.when(c == 0)
        def _():
            pltpu.sync_copy(pn0, pn_sp)

        @pl.when(c == 1)
        def _():
            pltpu.sync_copy(pn1, pn_sp)
        pltpu.sync_copy(node_hbm, node_sp)

    pltpu.sync_copy(src_hbm.at[pl.ds(s * EPT, EPT)], isrc)
    pltpu.sync_copy(dst_hbm.at[pl.ds(s * EPT, EPT)], idst)
    plsc.subcore_barrier()

    _agg_pipe(pn_sp, isrc, idst, rows, rows2, acc, sem_a, sem_b, deg, rdbuf,
              sem_d, node_sp, na, nb)

    plsc.subcore_barrier()

    pltpu.sync_copy(b1_hbm.at[pl.ds(c * HALF, HALF)], b1buf)

    def combine(ps_ref, hout_ref, do_rdeg):
        def blk(j, carry):
            b = s * BPT + j

            @pl.when(b < NBLK)
            def _():
                base = b * K
                pltpu.sync_copy(node_hbm.at[pl.ds(base, K)], nidx)

                @pl.when(j > 0)
                def _():
                    pltpu.make_async_copy(rows, hout_ref.at[pl.ds(0, K)],
                                          sem_d).wait()
                pltpu.async_copy(ps_ref.at[nidx], rows, sem_a)
                pltpu.async_copy(acc.at[pl.ds(base, K)], rows2, sem_b)
                pltpu.async_copy(deg.at[pl.ds(base, K)], degblk, sem_c)
                pltpu.make_async_copy(ps_ref.at[nidx], rows, sem_a).wait()
                pltpu.make_async_copy(acc.at[pl.ds(0, K)], rows2, sem_b).wait()
                pltpu.make_async_copy(deg.at[pl.ds(0, K)], degblk, sem_c).wait()

                def tfn(t, carry2):
                    dvec = degblk[pl.ds(t * LANES, LANES)]
                    rdvec = 1.0 / jnp.maximum(dvec, 1.0)
                    for l in range(LANES):
                        i = t * LANES + l
                        rd = lax.broadcast_in_dim(
                            lax.slice(rdvec, (l,), (l + 1,)), (LANES,), (0,))
                        for jj in range(HALF // LANES):
                            sl = pl.ds(jj * LANES, LANES)
                            v = rows[i, sl] + rows2[i, sl] * rd + b1buf[sl]
                            rows[i, sl] = jnp.maximum(v, 0.0)
                    return carry2
                lax.fori_loop(0, K // LANES, tfn, None)
                pltpu.async_copy(rows, hout_ref.at[pl.ds(base, K)], sem_d)
                if do_rdeg:
                    def rv(i, carry2):
                        sl = pl.ds(i * LANES, LANES)
                        rdbuf[sl] = 1.0 / jnp.maximum(degblk[sl], 1.0)
                        return carry2
                    lax.fori_loop(0, K // LANES, rv, None)
                    pltpu.sync_copy(rdbuf, rdeg_out.at[pl.ds(base, K)])
            return carry
        lax.fori_loop(0, BPT, blk, None)
        pltpu.make_async_copy(rows, hout_ref.at[pl.ds(0, K)], sem_d).wait()

    @pl.when(c == 0)
    def _():
        combine(ps0, h1a, True)

    @pl.when(c == 1)
    def _():
        combine(ps1, h1b, False)


def _sc_layer1(node, srcf, dstf, ps0, ps1, pn0, pn1, b1):
    mesh = plsc.VectorSubcoreMesh(core_axis_name="c", subcore_axis_name="s",
                                  num_cores=NC, num_subcores=NS)
    kern = pl.kernel(
        _sc1_body,
        out_type=[jax.ShapeDtypeStruct((N_NODES, HALF), f32),
                  jax.ShapeDtypeStruct((N_NODES, HALF), f32),
                  jax.ShapeDtypeStruct((N_NODES,), f32)],
        mesh=mesh,
        scratch_types=[
            pltpu.VMEM_SHARED((N_NODES, HALF), f32),
            pltpu.VMEM_SHARED((DEG_PAD,), f32),
            pltpu.VMEM_SHARED((VOCAB, HALF), f32),
            pltpu.VMEM_SHARED((N_NODES,), i32),
            pltpu.VMEM((EPT,), i32),
            pltpu.VMEM((EPT,), i32),
            pltpu.VMEM((K,), i32),
            pltpu.VMEM((K,), i32),
            pltpu.VMEM((K, HALF), f32),
            pltpu.VMEM((K, HALF), f32),
            pltpu.VMEM((K,), f32),
            pltpu.VMEM((HALF,), f32),
            pltpu.VMEM((K,), i32),
            pltpu.VMEM((K,), f32),
            pltpu.SemaphoreType.DMA,
            pltpu.SemaphoreType.DMA,
            pltpu.SemaphoreType.DMA,
            pltpu.SemaphoreType.DMA,
        ],
    )
    return kern(node, srcf, dstf, ps0, ps1, pn0, pn1, b1)


def _sc2_body(src_hbm, dst_hbm, s2a, s2b, g2a, g2b, rdeg_hbm, b2_hbm,
              h2_out,
              acc, isrc, idst, rows, rows2, degblk, b2buf, sem_a, sem_b,
              sem_c, sem_d):
    c = lax.axis_index("c")
    s = lax.axis_index("s")

    _zero_vmem_2d(rows, K)
    def zblk(q, carry):
        b = s * BPT + q

        @pl.when(b < NBLK)
        def _():
            pltpu.async_copy(rows, acc.at[pl.ds(b * K, K)], sem_a)
        return carry
    lax.fori_loop(0, BPT, zblk, None)
    def zdrain(q, carry):
        b = s * BPT + q

        @pl.when(b < NBLK)
        def _():
            pltpu.make_async_copy(rows, acc.at[pl.ds(0, K)], sem_a).wait()
        return carry
    lax.fori_loop(0, BPT, zdrain, None)
    plsc.subcore_barrier()

    pltpu.sync_copy(src_hbm.at[pl.ds(s * EPT, EPT)], isrc)
    pltpu.sync_copy(dst_hbm.at[pl.ds(s * EPT, EPT)], idst)

    @pl.when(c == 0)
    def _():
        _agg_pipe(g2a, isrc, idst, rows, rows2, acc, sem_a, sem_b)

    @pl.when(c == 1)
    def _():
        _agg_pipe(g2b, isrc, idst, rows, rows2, acc, sem_a, sem_b)

    plsc.subcore_barrier()

    pltpu.sync_copy(b2_hbm.at[pl.ds(c * HALF, HALF)], b2buf)

    def combine(s_ref):
        def blk(j, carry):
            b = s * BPT + j

            @pl.when(b < NBLK)
            def _():
                base = b * K

                @pl.when(j > 0)
                def _():
                    pltpu.make_async_copy(
                        rows, h2_out.at[pl.ds(0, K), pl.ds(0, HALF)],
                        sem_d).wait()
                pltpu.async_copy(s_ref.at[pl.ds(base, K)], rows, sem_a)
                pltpu.async_copy(acc.at[pl.ds(base, K)], rows2, sem_b)
                pltpu.async_copy(rdeg_hbm.at[pl.ds(base, K)], degblk, sem_c)
                pltpu.make_async_copy(s_ref.at[pl.ds(0, K)], rows, sem_a).wait()
                pltpu.make_async_copy(acc.at[pl.ds(0, K)], rows2, sem_b).wait()
                pltpu.make_async_copy(rdeg_hbm.at[pl.ds(0, K)], degblk,
                                      sem_c).wait()

                def tfn(t, carry2):
                    rdvec = degblk[pl.ds(t * LANES, LANES)]
                    for l in range(LANES):
                        i = t * LANES + l
                        rd = lax.broadcast_in_dim(
                            lax.slice(rdvec, (l,), (l + 1,)), (LANES,), (0,))
                        for jj in range(HALF // LANES):
                            sl = pl.ds(jj * LANES, LANES)
                            rows[i, sl] = (rows[i, sl] + rows2[i, sl] * rd
                                           + b2buf[sl])
                    return carry2
                lax.fori_loop(0, K // LANES, tfn, None)
                pltpu.async_copy(
                    rows, h2_out.at[pl.ds(base, K), pl.ds(c * HALF, HALF)],
                    sem_d)
            return carry
        lax.fori_loop(0, BPT, blk, None)
        pltpu.make_async_copy(rows, h2_out.at[pl.ds(0, K), pl.ds(0, HALF)],
                              sem_d).wait()

    @pl.when(c == 0)
    def _():
        combine(s2a)

    @pl.when(c == 1)
    def _():
        combine(s2b)


def _sc_layer2(srcf, dstf, s2a, s2b, g2a, g2b, rdeg, b2):
    mesh = plsc.VectorSubcoreMesh(core_axis_name="c", subcore_axis_name="s",
                                  num_cores=NC, num_subcores=NS)
    kern = pl.kernel(
        _sc2_body,
        out_type=jax.ShapeDtypeStruct((N_NODES, D_H), f32),
        mesh=mesh,
        scratch_types=[
            pltpu.VMEM_SHARED((N_NODES, HALF), f32),
            pltpu.VMEM((EPT,), i32),
            pltpu.VMEM((EPT,), i32),
            pltpu.VMEM((K, HALF), f32),
            pltpu.VMEM((K, HALF), f32),
            pltpu.VMEM((K,), f32),
            pltpu.VMEM((HALF,), f32),
            pltpu.SemaphoreType.DMA,
            pltpu.SemaphoreType.DMA,
            pltpu.SemaphoreType.DMA,
            pltpu.SemaphoreType.DMA,
        ],
    )
    return kern(srcf, dstf, s2a, s2b, g2a, g2b, rdeg, b2)


def kernel(node, edge_index, emb, W_self1, W_neigh1, b1, W_self2, W_neigh2, b2):
    srcf = edge_index[0]
    dstf = edge_index[1]
    ps0, ps1, pn0, pn1 = _project_l1(emb, W_self1, W_neigh1)
    h1a, h1b, rdeg = _sc_layer1(node, srcf, dstf, ps0, ps1, pn0, pn1, b1)
    s2a, s2b, g2a, g2b = _project_l2(h1a, h1b, W_self2, W_neigh2)
    return _sc_layer2(srcf, dstf, s2a, s2b, g2a, g2b, rdeg, b2)


# speedup vs baseline: 1.4229x; 1.1615x over previous
import jax
import jax.numpy as jnp
from jax import lax
from jax.experimental import pallas as pl
from jax.experimental.pallas import tpu as pltpu
from jax.experimental.pallas import tpu_sc as plsc

N_NODES = 10000
N_EDGES = 160000
VOCAB = 1000
D_IN = 1024
D_H = 256
HALF = 128
NC = 2
NS = 16
LANES = 16
K = 80
EPT = N_EDGES // NS
KCH = EPT // K
NBLK = N_NODES // K
BPT = (NBLK + NS - 1) // NS
ZROWS = 125
DEG_PAD = NS * 640

f32 = jnp.float32
i32 = jnp.int32


def _proj1_body(emb_ref, ws_ref, wn_ref, ps0_ref, ps1_ref, pn0_ref, pn1_ref):
    e = emb_ref[...]
    ps = jnp.dot(e, ws_ref[...], preferred_element_type=f32)
    pn = jnp.dot(e, wn_ref[...], preferred_element_type=f32)
    ps0_ref[...] = ps[:, :HALF]
    ps1_ref[...] = ps[:, HALF:]
    pn0_ref[...] = pn[:, :HALF]
    pn1_ref[...] = pn[:, HALF:]


def _project_l1(emb, W_self1, W_neigh1):
    out = jax.ShapeDtypeStruct((VOCAB, HALF), f32)
    return pl.pallas_call(_proj1_body, out_shape=(out,) * 4)(
        emb, W_self1, W_neigh1)


def _proj2_body(h1a_ref, h1b_ref, ws_ref, wn_ref, s0_ref, s1_ref, g0_ref, g1_ref):
    a = h1a_ref[...]
    b = h1b_ref[...]
    ws = ws_ref[...]
    wn = wn_ref[...]
    s = (jnp.dot(a, ws[:HALF, :], preferred_element_type=f32)
         + jnp.dot(b, ws[HALF:, :], preferred_element_type=f32))
    g = (jnp.dot(a, wn[:HALF, :], preferred_element_type=f32)
         + jnp.dot(b, wn[HALF:, :], preferred_element_type=f32))
    s0_ref[...] = s[:, :HALF]
    s1_ref[...] = s[:, HALF:]
    g0_ref[...] = g[:, :HALF]
    g1_ref[...] = g[:, HALF:]


def _project_l2(h1a, h1b, W_self2, W_neigh2):
    R = 1000
    bs_in = pl.BlockSpec((R, HALF), lambda i: (i, 0))
    bs_w = pl.BlockSpec((D_H, D_H), lambda i: (0, 0))
    bs_out = pl.BlockSpec((R, HALF), lambda i: (i, 0))
    out = jax.ShapeDtypeStruct((N_NODES, HALF), f32)
    return pl.pallas_call(
        _proj2_body,
        grid=(N_NODES // R,),
        in_specs=[bs_in, bs_in, bs_w, bs_w],
        out_specs=(bs_out,) * 4,
        out_shape=(out,) * 4,
    )(h1a, h1b, W_self2, W_neigh2)


def _zero_vmem_2d(ref, nrows):
    def zrow(i, carry):
        for jj in range(HALF // LANES):
            ref[i, pl.ds(jj * LANES, LANES)] = jnp.zeros((LANES,), f32)
        return carry
    lax.fori_loop(0, nrows, zrow, None)


def _fill_vmem_1d(ref, n, value):
    def fill(i, carry):
        ref[pl.ds(i * LANES, LANES)] = jnp.full((LANES,), value, f32)
        return carry
    lax.fori_loop(0, n // LANES, fill, None)


def _agg_pipe(tab_ref, isrc, idst, rows, rows2, acc, sem_a, sem_b,
              deg=None, rdbuf=None, sem_d=None,
              node_sp=None, na=None, nb=None):
    def gidx(k):
        return isrc.at[pl.ds(k * K, K)]

    def didx(k):
        return idst.at[pl.ds(k * K, K)]

    def scat(buf, k):
        pltpu.sync_copy(buf, acc.at[didx(k)], add=True)
        if deg is not None:
            pltpu.async_copy(rdbuf, deg.at[didx(k)], sem_d, add=True)

    def ng(k, nbuf):
        pltpu.sync_copy(node_sp.at[gidx(k)], nbuf)

    def rg(k, nbuf, buf, sem):
        if node_sp is None:
            pltpu.async_copy(tab_ref.at[gidx(k)], buf, sem)
        else:
            pltpu.async_copy(tab_ref.at[nbuf], buf, sem)

    def rwait(buf, sem):
        pltpu.make_async_copy(tab_ref.at[gidx(0)], buf, sem).wait()

    if node_sp is not None:
        ng(0, na)
    rg(0, na, rows, sem_a)

    def body(kk, carry):
        k0 = 2 * kk
        k1 = k0 + 1
        if node_sp is not None:
            ng(k1, nb)
        rwait(rows, sem_a)
        rg(k1, nb, rows2, sem_b)
        scat(rows, k0)
        if node_sp is not None:
            ng(k0 + 2, na)
        rwait(rows2, sem_b)
        rg(k0 + 2, na, rows, sem_a)
        scat(rows2, k1)
        return carry
    lax.fori_loop(0, (KCH - 1) // 2, body, None)
    rwait(rows, sem_a)
    scat(rows, KCH - 1)
    if deg is not None:
        def drain(k, carry):
            pltpu.make_async_copy(rdbuf, deg.at[didx(0)], sem_d).wait()
            return carry
        lax.fori_loop(0, KCH, drain, None)


def _sc1_body(node_hbm, src_hbm, dst_hbm, ps0, ps1, pn0, pn1, b1_hbm,
              h1a, h1b, rdeg_out,
              acc, deg, pn_sp, node_sp, isrc, idst, na, nb,
              rows, rows2, degblk, b1buf, nidx, rdbuf, sem_a, sem_b, sem_c,
              sem_d):
    c = lax.axis_index("c")
    s = lax.axis_index("s")

    _zero_vmem_2d(rows, K)
    _fill_vmem_1d(rdbuf, K, 0.0)
    def zblk(q, carry):
        b = s * BPT + q

        @pl.when(b < NBLK)
        def _():
            pltpu.async_copy(rows, acc.at[pl.ds(b * K, K)], sem_a)
        return carry
    lax.fori_loop(0, BPT, zblk, None)
    for q in range(8):
        pltpu.async_copy(rdbuf, deg.at[pl.ds(s * 640 + q * K, K)], sem_b)
    def zdrain(q, carry):
        b = s * BPT + q

        @pl.when(b < NBLK)
        def _():
            pltpu.make_async_copy(rows, acc.at[pl.ds(0, K)], sem_a).wait()
        return carry
    lax.fori_loop(0, BPT, zdrain, None)
    for q in range(8):
        pltpu.make_async_copy(rdbuf, deg.at[pl.ds(0, K)], sem_b).wait()
    _fill_vmem_1d(rdbuf, K, 1.0)

    @pl.when(s == 0)
    def _():
        @pl.when(c == 0)
        def _():
            pltpu.sync_copy(pn0, pn_sp)

        @pl.when(c == 1)
        def _():
            pltpu.sync_copy(pn1, pn_sp)
        pltpu.sync_copy(node_hbm, node_sp)

    pltpu.sync_copy(src_hbm.at[pl.ds(s * EPT, EPT)], isrc)
    pltpu.sync_copy(dst_hbm.at[pl.ds(s * EPT, EPT)], idst)
    plsc.subcore_barrier()

    _agg_pipe(pn_sp, isrc, idst, rows, rows2, acc, sem_a, sem_b, deg, rdbuf,
              sem_d, node_sp, na, nb)

    plsc.subcore_barrier()

    pltpu.sync_copy(b1_hbm.at[pl.ds(c * HALF, HALF)], b1buf)

    def combine(ps_ref, hout_ref, do_rdeg):
        def blk(j, carry):
            b = s * BPT + j

            @pl.when(b < NBLK)
            def _():
                base = b * K
                pltpu.sync_copy(node_hbm.at[pl.ds(base, K)], nidx)

                @pl.when(j > 0)
                def _():
                    pltpu.make_async_copy(rows, hout_ref.at[pl.ds(0, K)],
                                          sem_d).wait()
                pltpu.async_copy(ps_ref.at[nidx], rows, sem_a)
                pltpu.async_copy(acc.at[pl.ds(base, K)], rows2, sem_b)
                pltpu.async_copy(deg.at[pl.ds(base, K)], degblk, sem_c)
                pltpu.make_async_copy(ps_ref.at[nidx], rows, sem_a).wait()
                pltpu.make_async_copy(acc.at[pl.ds(0, K)], rows2, sem_b).wait()
                pltpu.make_async_copy(deg.at[pl.ds(0, K)], degblk, sem_c).wait()

                def tfn(t, carry2):
                    dvec = degblk[pl.ds(t * LANES, LANES)]
                    rdvec = 1.0 / jnp.maximum(dvec, 1.0)
                    for l in range(LANES):
                        i = t * LANES + l
                        rd = lax.broadcast_in_dim(
                            lax.slice(rdvec, (l,), (l + 1,)), (LANES,), (0,))
                        for jj in range(HALF // LANES):
                            sl = pl.ds(jj * LANES, LANES)
                            v = rows[i, sl] + rows2[i, sl] * rd + b1buf[sl]
                            rows[i, sl] = jnp.maximum(v, 0.0)
                    return carry2
                lax.fori_loop(0, K // LANES, tfn, None)
                pltpu.async_copy(rows, hout_ref.at[pl.ds(base, K)], sem_d)
                if do_rdeg:
                    def rv(i, carry2):
                        sl = pl.ds(i * LANES, LANES)
                        rdbuf[sl] = 1.0 / jnp.maximum(degblk[sl], 1.0)
                        return carry2
                    lax.fori_loop(0, K // LANES, rv, None)
                    pltpu.sync_copy(rdbuf, rdeg_out.at[pl.ds(base, K)])
            return carry
        lax.fori_loop(0, BPT, blk, None)
        pltpu.make_async_copy(rows, hout_ref.at[pl.ds(0, K)], sem_d).wait()

    @pl.when(c == 0)
    def _():
        combine(ps0, h1a, True)

    @pl.when(c == 1)
    def _():
        combine(ps1, h1b, False)


def _sc_layer1(node, srcf, dstf, ps0, ps1, pn0, pn1, b1):
    mesh = plsc.VectorSubcoreMesh(core_axis_name="c", subcore_axis_name="s",
                                  num_cores=NC, num_subcores=NS)
    kern = pl.kernel(
        _sc1_body,
        out_type=[jax.ShapeDtypeStruct((N_NODES, HALF), f32),
                  jax.ShapeDtypeStruct((N_NODES, HALF), f32),
                  jax.ShapeDtypeStruct((N_NODES,), f32)],
        mesh=mesh,
        scratch_types=[
            pltpu.VMEM_SHARED((N_NODES, HALF), f32),
            pltpu.VMEM_SHARED((DEG_PAD,), f32),
            pltpu.VMEM_SHARED((VOCAB, HALF), f32),
            pltpu.VMEM_SHARED((N_NODES,), i32),
            pltpu.VMEM((EPT,), i32),
            pltpu.VMEM((EPT,), i32),
            pltpu.VMEM((K,), i32),
            pltpu.VMEM((K,), i32),
            pltpu.VMEM((K, HALF), f32),
            pltpu.VMEM((K, HALF), f32),
            pltpu.VMEM((K,), f32),
            pltpu.VMEM((HALF,), f32),
            pltpu.VMEM((K,), i32),
            pltpu.VMEM((K,), f32),
            pltpu.SemaphoreType.DMA,
            pltpu.SemaphoreType.DMA,
            pltpu.SemaphoreType.DMA,
            pltpu.SemaphoreType.DMA,
        ],
    )
    return kern(node, srcf, dstf, ps0, ps1, pn0, pn1, b1)


def _sc2_body(src_hbm, dst_hbm, s2a, s2b, g2a, g2b, rdeg_hbm, b2_hbm,
              h2_out,
              acc, isrc, idst, rows, rows2, rows3, degblk, b2buf, sem_a,
              sem_b, sem_c, sem_d):
    c = lax.axis_index("c")
    s = lax.axis_index("s")

    _zero_vmem_2d(rows, K)
    def zblk(q, carry):
        b = s * BPT + q

        @pl.when(b < NBLK)
        def _():
            pltpu.async_copy(rows, acc.at[pl.ds(b * K, K)], sem_a)
        return carry
    lax.fori_loop(0, BPT, zblk, None)
    def zdrain(q, carry):
        b = s * BPT + q

        @pl.when(b < NBLK)
        def _():
            pltpu.make_async_copy(rows, acc.at[pl.ds(0, K)], sem_a).wait()
        return carry
    lax.fori_loop(0, BPT, zdrain, None)
    plsc.subcore_barrier()

    def agg3(tab):
        bufs = (rows, rows2, rows3)
        sems = (sem_a, sem_b, sem_c)

        def lidx(q, k):
            return q.at[pl.ds(k * K, K)]

        for n, seg_base in ((63, 0), (62, 63)):
            pltpu.sync_copy(
                src_hbm.at[pl.ds(s * EPT + seg_base * K, n * K)],
                isrc.at[pl.ds(0, n * K)])
            pltpu.sync_copy(
                dst_hbm.at[pl.ds(s * EPT + seg_base * K, n * K)],
                idst.at[pl.ds(0, n * K)])
            pltpu.async_copy(tab.at[lidx(isrc, 0)], bufs[0], sems[0])
            pltpu.async_copy(tab.at[lidx(isrc, 1)], bufs[1], sems[1])

            def triple(t, carry):
                for r in range(3):
                    k = 3 * t + r
                    nxt = k + 2
                    rn = (r + 2) % 3

                    @pl.when(nxt < n)
                    def _():
                        pltpu.async_copy(tab.at[lidx(isrc, nxt)], bufs[rn],
                                         sems[rn])
                    pltpu.make_async_copy(tab.at[lidx(isrc, 0)], bufs[r],
                                          sems[r]).wait()
                    pltpu.sync_copy(bufs[r], acc.at[lidx(idst, k)], add=True)
                return carry
            lax.fori_loop(0, n // 3, triple, None)
            for k in range(3 * (n // 3), n):
                r = k % 3
                pltpu.make_async_copy(tab.at[lidx(isrc, 0)], bufs[r],
                                      sems[r]).wait()
                pltpu.sync_copy(bufs[r], acc.at[lidx(idst, k)], add=True)

    @pl.when(c == 0)
    def _():
        agg3(g2a)

    @pl.when(c == 1)
    def _():
        agg3(g2b)

    plsc.subcore_barrier()

    pltpu.sync_copy(b2_hbm.at[pl.ds(c * HALF, HALF)], b2buf)

    def combine(s_ref):
        def blk(j, carry):
            b = s * BPT + j

            @pl.when(b < NBLK)
            def _():
                base = b * K

                @pl.when(j > 0)
                def _():
                    pltpu.make_async_copy(
                        rows, h2_out.at[pl.ds(0, K), pl.ds(0, HALF)],
                        sem_d).wait()
                pltpu.async_copy(s_ref.at[pl.ds(base, K)], rows, sem_a)
                pltpu.async_copy(acc.at[pl.ds(base, K)], rows2, sem_b)
                pltpu.async_copy(rdeg_hbm.at[pl.ds(base, K)], degblk, sem_c)
                pltpu.make_async_copy(s_ref.at[pl.ds(0, K)], rows, sem_a).wait()
                pltpu.make_async_copy(acc.at[pl.ds(0, K)], rows2, sem_b).wait()
                pltpu.make_async_copy(rdeg_hbm.at[pl.ds(0, K)], degblk,
                                      sem_c).wait()

                def tfn(t, carry2):
                    rdvec = degblk[pl.ds(t * LANES, LANES)]
                    for l in range(LANES):
                        i = t * LANES + l
                        rd = lax.broadcast_in_dim(
                            lax.slice(rdvec, (l,), (l + 1,)), (LANES,), (0,))
                        for jj in range(HALF // LANES):
                            sl = pl.ds(jj * LANES, LANES)
                            rows[i, sl] = (rows[i, sl] + rows2[i, sl] * rd
                                           + b2buf[sl])
                    return carry2
                lax.fori_loop(0, K // LANES, tfn, None)
                pltpu.async_copy(
                    rows, h2_out.at[pl.ds(base, K), pl.ds(c * HALF, HALF)],
                    sem_d)
            return carry
        lax.fori_loop(0, BPT, blk, None)
        pltpu.make_async_copy(rows, h2_out.at[pl.ds(0, K), pl.ds(0, HALF)],
                              sem_d).wait()

    @pl.when(c == 0)
    def _():
        combine(s2a)

    @pl.when(c == 1)
    def _():
        combine(s2b)


def _sc_layer2(srcf, dstf, s2a, s2b, g2a, g2b, rdeg, b2):
    mesh = plsc.VectorSubcoreMesh(core_axis_name="c", subcore_axis_name="s",
                                  num_cores=NC, num_subcores=NS)
    kern = pl.kernel(
        _sc2_body,
        out_type=jax.ShapeDtypeStruct((N_NODES, D_H), f32),
        mesh=mesh,
        scratch_types=[
            pltpu.VMEM_SHARED((N_NODES, HALF), f32),
            pltpu.VMEM((5040,), i32),
            pltpu.VMEM((5040,), i32),
            pltpu.VMEM((K, HALF), f32),
            pltpu.VMEM((K, HALF), f32),
            pltpu.VMEM((K, HALF), f32),
            pltpu.VMEM((K,), f32),
            pltpu.VMEM((HALF,), f32),
            pltpu.SemaphoreType.DMA,
            pltpu.SemaphoreType.DMA,
            pltpu.SemaphoreType.DMA,
            pltpu.SemaphoreType.DMA,
        ],
    )
    return kern(srcf, dstf, s2a, s2b, g2a, g2b, rdeg, b2)


def kernel(node, edge_index, emb, W_self1, W_neigh1, b1, W_self2, W_neigh2, b2):
    srcf = edge_index[0]
    dstf = edge_index[1]
    ps0, ps1, pn0, pn1 = _project_l1(emb, W_self1, W_neigh1)
    h1a, h1b, rdeg = _sc_layer1(node, srcf, dstf, ps0, ps1, pn0, pn1, b1)
    s2a, s2b, g2a, g2b = _project_l2(h1a, h1b, W_self2, W_neigh2)
    return _sc_layer2(srcf, dstf, s2a, s2b, g2a, g2b, rdeg, b2)
